# baseline (speedup 1.0000x reference)
import jax
import jax.numpy as jnp
from jax.experimental import pallas as pl

N = 10000
E = 320000
D = 128
H = 128
K = 3
C = 2


def kernel(x, conv_w, conv_b, W_w, W_b, attn_w, attn_b, Wk_w, Wk_b, lin_w, lin_b, edge_index, motif_mask):
    src = edge_index[0]
    dst = edge_index[1]
    loop = jnp.arange(N, dtype=src.dtype)
    s2 = jnp.concatenate([src, loop])
    d2 = jnp.concatenate([dst, loop])
    ones = jnp.ones((s2.shape[0],), dtype=jnp.float32)
    deg_out = jax.ops.segment_sum(ones, s2, N)
    deg_in = jax.ops.segment_sum(ones, d2, N)
    norm_s = jax.lax.rsqrt(jnp.maximum(deg_out, 1.0))
    norm_d = jax.lax.rsqrt(jnp.maximum(deg_in, 1.0))
    msg = x[s2] * norm_s[s2][:, None]
    agg = jax.ops.segment_sum(msg, d2, N)
    hconv = (agg * norm_d[:, None]) @ conv_w + conv_b
    Z = hconv @ W_w + W_b
    pooled = [jnp.mean(Z, axis=0)]
    mm = motif_mask.astype(jnp.float32)
    e_all = jax.nn.leaky_relu((jnp.concatenate([Z[src], Z[dst]], axis=1) @ attn_w + attn_b)[:, 0])
    for k in range(K):
        az = Z @ Wk_w[k] + Wk_b[k]
        mask = mm[k]
        elog = jnp.where(mask > 0, e_all, -1e9)
        mx = jax.ops.segment_max(elog, dst, N)
        ee = jnp.exp(elog - mx[dst]) * mask
        den = jax.ops.segment_sum(ee, dst, N)
        alpha = ee / (den[dst] + 1e-9)
        hk = jax.ops.segment_sum(alpha[:, None] * az[src], dst, N)
        pooled.append(jnp.mean(hk, axis=0))
    hcat = jnp.concatenate(pooled, axis=0)
    out = hcat @ lin_w + lin_b
    return out[None, :]


# speedup vs baseline: 62.4884x; 62.4884x over previous
import functools

import jax
import jax.numpy as jnp
from jax import lax
from jax.experimental import pallas as pl
from jax.experimental.pallas import tpu as pltpu
from jax.experimental.pallas import tpu_sc as plsc

N = 10000
E = 320000
D = 128
H = 128
K = 3
C = 2

NC = 2
NS = 16
NW = NC * NS
L = 16
CH = 128
EP = 327680
NCH2 = EP // CH
CPT = NCH2 // NW
NPAD = 10240
SLICE = NPAD // NS
BLK = 256
NBLK = NPAD // BLK

_mesh = plsc.VectorSubcoreMesh(core_axis_name="c", subcore_axis_name="s")
_sc_params = pltpu.CompilerParams(needs_layout_passes=False)


def _wid():
    return lax.axis_index("s") * NC + lax.axis_index("c")


@functools.partial(
    pl.kernel,
    out_type=jax.ShapeDtypeStruct((NC * 2 * NPAD,), jnp.float32),
    mesh=_mesh,
    compiler_params=_sc_params,
    scratch_types=[
        pltpu.VMEM((CH,), jnp.float32),
        pltpu.VMEM((2, CH), jnp.int32),
        pltpu.VMEM((2, CH), jnp.int32),
        pltpu.VMEM((2, CH), jnp.int32),
        pltpu.VMEM((2, CH), jnp.int32),
        pltpu.SemaphoreType.DMA,
        pltpu.SemaphoreType.DMA,
        pltpu.SemaphoreType.DMA,
        pltpu.SemaphoreType.DMA,
        pltpu.SemaphoreType.DMA,
        pltpu.SemaphoreType.DMA,
        pltpu.MemorySpace.VMEM_SHARED((NPAD,), jnp.float32),
        pltpu.MemorySpace.VMEM_SHARED((NPAD,), jnp.float32),
    ],
)
def _sc_deg(e2_h, ones_h, zeros1_h, out_h, ones_v, e0, e1, e2b, e3,
            is0, is1, is2, is3, os0, os1, dego_sh, degi_sh):
    c = lax.axis_index("c")
    s = lax.axis_index("s")
    wid = _wid()
    start = s * SLICE
    ebufs = (e0, e1, e2b, e3)
    isems = (is0, is1, is2, is3)
    osems = (os0, os1)

    def issue_in(t, b):
        pltpu.async_copy(e2_h.at[wid + NW * t], ebufs[b], isems[b])

    def wait_in(b):
        pltpu.make_async_copy(e2_h.at[0], ebufs[b], isems[b]).wait()

    def issue_out(b, p):
        pltpu.async_copy(ones_v, dego_sh.at[ebufs[b].at[0]], osems[p],
                         add=True)
        pltpu.async_copy(ones_v, degi_sh.at[ebufs[b].at[1]], osems[p],
                         add=True)

    def wait_out(b, p):
        pltpu.make_async_copy(ones_v, dego_sh.at[ebufs[b].at[0]],
                              osems[p]).wait()
        pltpu.make_async_copy(ones_v, degi_sh.at[ebufs[b].at[1]],
                              osems[p]).wait()

    pltpu.sync_copy(zeros1_h, dego_sh.at[pl.ds(start, SLICE)])
    pltpu.sync_copy(zeros1_h, degi_sh.at[pl.ds(start, SLICE)])
    pltpu.sync_copy(ones_h, ones_v)
    issue_in(0, 0)
    issue_in(1, 1)
    plsc.subcore_barrier()

    def outer(i, carry):
        t0 = i * 4
        for b in range(4):
            t = t0 + b
            wait_in(b)

            @pl.when(t >= 2)
            def _(b=b):
                wait_out((b + 2) % 4, b % 2)

            @pl.when(t + 2 < CPT)
            def _(t=t, b=b):
                issue_in(t + 2, (b + 2) % 4)

            issue_out(b, b % 2)
        return carry

    lax.fori_loop(0, CPT // 4, outer, 0)
    wait_out(2, 0)
    wait_out(3, 1)
    plsc.subcore_barrier()
    pltpu.sync_copy(dego_sh.at[pl.ds(start, SLICE)],
                    out_h.at[pl.ds((c * 2 + 0) * NPAD + start, SLICE)])
    pltpu.sync_copy(degi_sh.at[pl.ds(start, SLICE)],
                    out_h.at[pl.ds((c * 2 + 1) * NPAD + start, SLICE)])


def _tc2_body(degparts, x, xn_out, ndcol_out):
    dsum = jnp.sum(degparts[...], axis=0) + 1.0
    norm = lax.rsqrt(jnp.maximum(dsum, 1.0))
    ident = (lax.broadcasted_iota(jnp.int32, (BLK, BLK), 0)
             == lax.broadcasted_iota(jnp.int32, (BLK, BLK), 1)
             ).astype(jnp.float32)
    dn = (((1,), (1,)), ((), ()))
    ncol_s = lax.dot_general(ident, norm[0:1, :], dn,
                             preferred_element_type=jnp.float32)
    ncol_d = lax.dot_general(ident, norm[1:2, :], dn,
                             preferred_element_type=jnp.float32)
    xn_out[...] = x[...] * ncol_s
    ndcol_out[...] = ncol_d


def _tc2(degparts, x_pad):
    return pl.pallas_call(
        _tc2_body,
        grid=(NBLK,),
        in_specs=[
            pl.BlockSpec((NC, 2, BLK), lambda i: (0, 0, i)),
            pl.BlockSpec((BLK, D), lambda i: (i, 0)),
        ],
        out_specs=[
            pl.BlockSpec((BLK, D), lambda i: (i, 0)),
            pl.BlockSpec((BLK, 1), lambda i: (i, 0)),
        ],
        out_shape=[
            jax.ShapeDtypeStruct((NPAD, D), jnp.float32),
            jax.ShapeDtypeStruct((NPAD, 1), jnp.float32),
        ],
    )(degparts, x_pad)


@functools.partial(
    pl.kernel,
    out_type=jax.ShapeDtypeStruct((NC, NPAD, D), jnp.float32),
    mesh=_mesh,
    compiler_params=_sc_params,
    scratch_types=[
        pltpu.VMEM((2, CH), jnp.int32),
        pltpu.VMEM((2, CH), jnp.int32),
        pltpu.VMEM((2, CH), jnp.int32),
        pltpu.VMEM((2, CH), jnp.int32),
        pltpu.VMEM((CH, D), jnp.float32),
        pltpu.VMEM((CH, D), jnp.float32),
        pltpu.SemaphoreType.DMA,
        pltpu.SemaphoreType.DMA,
        pltpu.SemaphoreType.DMA,
        pltpu.SemaphoreType.DMA,
        pltpu.SemaphoreType.DMA,
        pltpu.SemaphoreType.DMA,
        pltpu.SemaphoreType.DMA,
        pltpu.SemaphoreType.DMA,
        pltpu.MemorySpace.VMEM_SHARED((NPAD, D), jnp.float32),
    ],
)
def _sc_agg(xn_h, e2_h, zrows_h, out_h, e0, e1, e2b, e3, r0, r1,
            is0, is1, is2, is3, gs0, gs1, ss0, ss1, acc_sh):
    c = lax.axis_index("c")
    s = lax.axis_index("s")
    wid = _wid()
    start = s * SLICE
    ebufs = (e0, e1, e2b, e3)
    rows = (r0, r1)
    isems = (is0, is1, is2, is3)
    gsems = (gs0, gs1)
    ssems = (ss0, ss1)

    def issue_in(t, b):
        pltpu.async_copy(e2_h.at[wid + NW * t], ebufs[b], isems[b])

    def wait_in(b):
        pltpu.make_async_copy(e2_h.at[0], ebufs[b], isems[b]).wait()

    def issue_gather(b, p):
        pltpu.async_copy(xn_h.at[ebufs[b].at[0]], rows[p], gsems[p])

    def wait_gather(b, p):
        pltpu.make_async_copy(xn_h.at[ebufs[b].at[0]], rows[p],
                              gsems[p]).wait()

    def issue_scatter(b, p):
        pltpu.async_copy(rows[p], acc_sh.at[ebufs[b].at[1]], ssems[p],
                         add=True)

    def wait_scatter(b, p):
        pltpu.make_async_copy(rows[p], acc_sh.at[ebufs[b].at[1]],
                              ssems[p]).wait()

    pltpu.sync_copy(zrows_h, acc_sh.at[pl.ds(start, SLICE)])
    for b in range(4):
        issue_in(b, b)
    plsc.subcore_barrier()
    wait_in(0)
    issue_gather(0, 0)

    def outer(i, carry):
        t0 = i * 4
        for b in range(4):
            t = t0 + b

            @pl.when(t + 1 < CPT)
            def _(t=t, b=b):
                wait_in((b + 1) % 4)

                @pl.when(t >= 1)
                def _(t=t, b=b):
                    wait_scatter((b + 3) % 4, (b + 1) % 2)

                    @pl.when(t + 3 < CPT)
                    def _(t=t, b=b):
                        issue_in(t + 3, (b + 3) % 4)

                issue_gather((b + 1) % 4, (b + 1) % 2)

            wait_gather(b, b % 2)
            issue_scatter(b, b % 2)
        return carry

    lax.fori_loop(0, CPT // 4, outer, 0)
    wait_scatter(2, 0)
    wait_scatter(3, 1)
    plsc.subcore_barrier()
    pltpu.sync_copy(acc_sh.at[pl.ds(start, SLICE)],
                    out_h.at[c, pl.ds(start, SLICE)])


def _tc4_body(aggparts, xn, ndcol, conv_w, conv_b, W_w, W_b, aw2,
              z_out, uv_out, zsum_out, muv_out):
    i = pl.program_id(0)
    A = (aggparts[0] + aggparts[1] + xn[...]) * ndcol[...]
    h = jnp.dot(A, conv_w[...], preferred_element_type=jnp.float32) + conv_b[...]
    Z = jnp.dot(h, W_w[...], preferred_element_type=jnp.float32) + W_b[...]
    UV = jnp.dot(Z, aw2[...], preferred_element_type=jnp.float32)
    z_out[...] = Z
    uv_out[...] = UV
    rows = lax.broadcasted_iota(jnp.int32, (BLK, 1), 0) + i * BLK
    valid = rows < N
    zs = jnp.sum(jnp.where(valid, Z, 0.0), axis=0, keepdims=True)
    mu = jnp.max(jnp.where(valid, UV, -3e38), axis=0, keepdims=True)

    @pl.when(i == 0)
    def _():
        zsum_out[...] = zs
        muv_out[...] = mu

    @pl.when(i > 0)
    def _():
        zsum_out[...] = zsum_out[...] + zs
        muv_out[...] = jnp.maximum(muv_out[...], mu)


def _tc4(aggparts, xn, ndcol, conv_w, conv_b, W_w, W_b, aw2):
    return pl.pallas_call(
        _tc4_body,
        grid=(NBLK,),
        in_specs=[
            pl.BlockSpec((NC, BLK, D), lambda i: (0, i, 0)),
            pl.BlockSpec((BLK, D), lambda i: (i, 0)),
            pl.BlockSpec((BLK, 1), lambda i: (i, 0)),
            pl.BlockSpec((D, D), lambda i: (0, 0)),
            pl.BlockSpec((1, D), lambda i: (0, 0)),
            pl.BlockSpec((D, H), lambda i: (0, 0)),
            pl.BlockSpec((1, H), lambda i: (0, 0)),
            pl.BlockSpec((H, 2), lambda i: (0, 0)),
        ],
        out_specs=[
            pl.BlockSpec((BLK, H), lambda i: (i, 0)),
            pl.BlockSpec((BLK, 2), lambda i: (i, 0)),
            pl.BlockSpec((1, H), lambda i: (0, 0)),
            pl.BlockSpec((1, 2), lambda i: (0, 0)),
        ],
        out_shape=[
            jax.ShapeDtypeStruct((NPAD, H), jnp.float32),
            jax.ShapeDtypeStruct((NPAD, 2), jnp.float32),
            jax.ShapeDtypeStruct((1, H), jnp.float32),
            jax.ShapeDtypeStruct((1, 2), jnp.float32),
        ],
    )(aggparts, xn, ndcol, conv_w, conv_b, W_w, W_b, aw2)


@functools.partial(
    pl.kernel,
    out_type=jax.ShapeDtypeStruct((NC * K * NPAD,), jnp.float32),
    mesh=_mesh,
    compiler_params=_sc_params,
    scratch_types=[
        pltpu.VMEM((NPAD,), jnp.float32),
        pltpu.VMEM((NPAD,), jnp.float32),
        pltpu.VMEM((L,), jnp.float32),
        pltpu.VMEM((L,), jnp.float32),
        pltpu.VMEM((5, CH), jnp.int32),
        pltpu.VMEM((5, CH), jnp.int32),
        pltpu.VMEM((5, CH), jnp.int32),
        pltpu.VMEM((5, CH), jnp.int32),
        pltpu.VMEM((K, CH), jnp.float32),
        pltpu.VMEM((K, CH), jnp.float32),
        pltpu.SemaphoreType.DMA,
        pltpu.SemaphoreType.DMA,
        pltpu.SemaphoreType.DMA,
        pltpu.SemaphoreType.DMA,
        pltpu.SemaphoreType.DMA,
        pltpu.SemaphoreType.DMA,
        pltpu.MemorySpace.VMEM_SHARED((NPAD,), jnp.float32),
        pltpu.MemorySpace.VMEM_SHARED((NPAD,), jnp.float32),
        pltpu.MemorySpace.VMEM_SHARED((NPAD,), jnp.float32),
    ],
)
def _sc_den(u_h, v_h, m_h, b_h, e5_h, zeros1_h, out_h,
            u_v, v_v, m_v, b_v, e0, e1, e2b, e3, p0, p1,
            is0, is1, is2, is3, os0, os1, d0, d1, d2):
    c = lax.axis_index("c")
    s = lax.axis_index("s")
    wid = _wid()
    start = s * SLICE
    ebufs = (e0, e1, e2b, e3)
    pbufs = (p0, p1)
    isems = (is0, is1, is2, is3)
    osems = (os0, os1)
    dens = (d0, d1, d2)

    def issue_in(t, b):
        pltpu.async_copy(e5_h.at[wid + NW * t], ebufs[b], isems[b])

    def wait_in(b):
        pltpu.make_async_copy(e5_h.at[0], ebufs[b], isems[b]).wait()

    def issue_out(b, p):
        for k in range(K):
            pltpu.async_copy(pbufs[p].at[k], dens[k].at[ebufs[b].at[1]],
                             osems[p], add=True)

    def wait_out(b, p):
        for k in range(K):
            pltpu.make_async_copy(pbufs[p].at[k],
                                  dens[k].at[ebufs[b].at[1]],
                                  osems[p]).wait()

    for dsh in dens:
        pltpu.sync_copy(zeros1_h, dsh.at[pl.ds(start, SLICE)])
    pltpu.sync_copy(u_h, u_v)
    pltpu.sync_copy(v_h, v_v)
    pltpu.sync_copy(m_h, m_v)
    pltpu.sync_copy(b_h, b_v)
    issue_in(0, 0)
    issue_in(1, 1)
    plsc.subcore_barrier()
    Mv = m_v[...]
    Bv = b_v[...]

    def outer(i, carry):
        t0 = i * 4
        for b in range(4):
            t = t0 + b
            wait_in(b)

            @pl.when(t >= 2)
            def _(b=b):
                wait_out((b + 2) % 4, b % 2)

            @pl.when(t + 2 < CPT)
            def _(t=t, b=b):
                issue_in(t + 2, (b + 2) % 4)

            eb = ebufs[b]
            pb = pbufs[b % 2]

            def inner(j, icarry, eb=eb, pb=pb):
                sl = pl.ds(j * L, L)
                si = eb[0, sl]
                di = eb[1, sl]
                uu = plsc.load_gather(u_v, [si])
                vv = plsc.load_gather(v_v, [di])
                t0v = uu + vv + Bv
                e = jnp.where(t0v > 0, t0v, t0v * jnp.float32(0.01))
                p = jnp.exp(e - Mv)
                for k in range(K):
                    pb[k, sl] = p * eb[2 + k, sl].astype(jnp.float32)
                return icarry

            lax.fori_loop(0, CH // L, inner, 0)
            issue_out(b, b % 2)
        return carry

    lax.fori_loop(0, CPT // 4, outer, 0)
    wait_out(2, 0)
    wait_out(3, 1)
    plsc.subcore_barrier()
    for k, dsh in enumerate(dens):
        pltpu.sync_copy(dsh.at[pl.ds(start, SLICE)],
                        out_h.at[pl.ds((c * K + k) * NPAD + start, SLICE)])


@functools.partial(
    pl.kernel,
    out_type=jax.ShapeDtypeStruct((NC * K * NPAD,), jnp.float32),
    mesh=_mesh,
    compiler_params=_sc_params,
    scratch_types=[
        pltpu.VMEM((NPAD,), jnp.float32),
        pltpu.VMEM((NPAD,), jnp.float32),
        pltpu.VMEM((L,), jnp.float32),
        pltpu.VMEM((L,), jnp.float32),
        pltpu.VMEM((NPAD,), jnp.float32),
        pltpu.VMEM((NPAD,), jnp.float32),
        pltpu.VMEM((NPAD,), jnp.float32),
        pltpu.VMEM((NPAD,), jnp.float32),
        pltpu.VMEM((5, CH), jnp.int32),
        pltpu.VMEM((5, CH), jnp.int32),
        pltpu.VMEM((5, CH), jnp.int32),
        pltpu.VMEM((5, CH), jnp.int32),
        pltpu.VMEM((K, CH), jnp.float32),
        pltpu.VMEM((K, CH), jnp.float32),
        pltpu.SemaphoreType.DMA,
        pltpu.SemaphoreType.DMA,
        pltpu.SemaphoreType.DMA,
        pltpu.SemaphoreType.DMA,
        pltpu.SemaphoreType.DMA,
        pltpu.SemaphoreType.DMA,
        pltpu.MemorySpace.VMEM_SHARED((NPAD,), jnp.float32),
        pltpu.MemorySpace.VMEM_SHARED((NPAD,), jnp.float32),
        pltpu.MemorySpace.VMEM_SHARED((NPAD,), jnp.float32),
    ],
)
def _sc_w(u_h, v_h, m_h, b_h, e5_h, denparts_h, zeros1_h, out_h,
          u_v, v_v, m_v, b_v, dn0, dn1, dn2, tmp, e0, e1, e2b, e3, a0, a1,
          is0, is1, is2, is3, os0, os1, w0, w1, w2):
    c = lax.axis_index("c")
    s = lax.axis_index("s")
    wid = _wid()
    start = s * SLICE
    ebufs = (e0, e1, e2b, e3)
    abufs = (a0, a1)
    isems = (is0, is1, is2, is3)
    osems = (os0, os1)
    ws = (w0, w1, w2)
    dns = (dn0, dn1, dn2)

    def issue_in(t, b):
        pltpu.async_copy(e5_h.at[wid + NW * t], ebufs[b], isems[b])

    def wait_in(b):
        pltpu.make_async_copy(e5_h.at[0], ebufs[b], isems[b]).wait()

    def issue_out(b, p):
        for k in range(K):
            pltpu.async_copy(abufs[p].at[k], ws[k].at[ebufs[b].at[0]],
                             osems[p], add=True)

    def wait_out(b, p):
        for k in range(K):
            pltpu.make_async_copy(abufs[p].at[k],
                                  ws[k].at[ebufs[b].at[0]],
                                  osems[p]).wait()

    for wsh in ws:
        pltpu.sync_copy(zeros1_h, wsh.at[pl.ds(start, SLICE)])
    pltpu.sync_copy(u_h, u_v)
    pltpu.sync_copy(v_h, v_v)
    pltpu.sync_copy(m_h, m_v)
    pltpu.sync_copy(b_h, b_v)
    issue_in(0, 0)
    issue_in(1, 1)
    for k, dn in enumerate(dns):
        pltpu.sync_copy(denparts_h.at[pl.ds(k * NPAD, NPAD)], dn)
        pltpu.sync_copy(denparts_h.at[pl.ds((K + k) * NPAD, NPAD)], tmp)

        def dsum(j, carry, dn=dn):
            sl = pl.ds(j * L, L)
            dn[sl] = dn[sl] + tmp[sl]
            return carry

        lax.fori_loop(0, NPAD // L, dsum, 0)
    plsc.subcore_barrier()
    Mv = m_v[...]
    Bv = b_v[...]

    def outer(i, carry):
        t0 = i * 4
        for b in range(4):
            t = t0 + b
            wait_in(b)

            @pl.when(t >= 2)
            def _(b=b):
                wait_out((b + 2) % 4, b % 2)

            @pl.when(t + 2 < CPT)
            def _(t=t, b=b):
                issue_in(t + 2, (b + 2) % 4)

            eb = ebufs[b]
            ab = abufs[b % 2]

            def inner(j, icarry, eb=eb, ab=ab):
                sl = pl.ds(j * L, L)
                si = eb[0, sl]
                di = eb[1, sl]
                uu = plsc.load_gather(u_v, [si])
                vv = plsc.load_gather(v_v, [di])
                t0v = uu + vv + Bv
                e = jnp.where(t0v > 0, t0v, t0v * jnp.float32(0.01))
                p = jnp.exp(e - Mv)
                for k in range(K):
                    dd = plsc.load_gather(dns[k], [di])
                    mk = eb[2 + k, sl].astype(jnp.float32)
                    ab[k, sl] = (p * mk) / (dd + jnp.float32(1e-30))
                return icarry

            lax.fori_loop(0, CH // L, inner, 0)
            issue_out(b, b % 2)
        return carry

    lax.fori_loop(0, CPT // 4, outer, 0)
    wait_out(2, 0)
    wait_out(3, 1)
    plsc.subcore_barrier()
    for k, wsh in enumerate(ws):
        pltpu.sync_copy(wsh.at[pl.ds(start, SLICE)],
                        out_h.at[pl.ds((c * K + k) * NPAD + start, SLICE)])


def _tc7_body(wparts, z, zsum, wkw, wkb, lw4, lb, out, s_acc, a_acc):
    i = pl.program_id(0)
    wm = wparts[0] + wparts[1]
    sblk = lax.dot_general(wm, z[...], (((1,), (0,)), ((), ())),
                           preferred_element_type=jnp.float32)
    ablk = jnp.sum(wm, axis=1, keepdims=True)

    @pl.when(i == 0)
    def _():
        s_acc[...] = sblk
        a_acc[...] = ablk

    @pl.when(i > 0)
    def _():
        s_acc[...] = s_acc[...] + sblk
        a_acc[...] = a_acc[...] + ablk

    @pl.when(i == NBLK - 1)
    def _():
        acc = jnp.dot(zsum[...], lw4[0], preferred_element_type=jnp.float32)
        for k in range(K):
            pk = (jnp.dot(s_acc[k:k + 1, :], wkw[k],
                          preferred_element_type=jnp.float32)
                  + a_acc[k:k + 1, 0:1] * wkb[k:k + 1, :])
            acc = acc + jnp.dot(pk, lw4[k + 1],
                                preferred_element_type=jnp.float32)
        out[...] = acc / jnp.float32(N) + lb[...]


def _tc7(wparts, z, zsum, wkw, wkb, lw4, lb):
    return pl.pallas_call(
        _tc7_body,
        grid=(NBLK,),
        in_specs=[
            pl.BlockSpec((NC, K, BLK), lambda i: (0, 0, i)),
            pl.BlockSpec((BLK, H), lambda i: (i, 0)),
            pl.BlockSpec((1, H), lambda i: (0, 0)),
            pl.BlockSpec((K, H, H), lambda i: (0, 0, 0)),
            pl.BlockSpec((K, H), lambda i: (0, 0)),
            pl.BlockSpec((K + 1, H, C), lambda i: (0, 0, 0)),
            pl.BlockSpec((1, C), lambda i: (0, 0)),
        ],
        out_specs=pl.BlockSpec((1, C), lambda i: (0, 0)),
        out_shape=jax.ShapeDtypeStruct((1, C), jnp.float32),
        scratch_shapes=[
            pltpu.VMEM((K, H), jnp.float32),
            pltpu.VMEM((K, 1), jnp.float32),
        ],
    )(wparts, z, zsum, wkw, wkb, lw4, lb)


def kernel(x, conv_w, conv_b, W_w, W_b, attn_w, attn_b, Wk_w, Wk_b,
           lin_w, lin_b, edge_index, motif_mask):
    pade = EP - E
    padi = (N + jnp.arange(pade, dtype=jnp.int32) % (NPAD - N)).astype(
        jnp.int32)
    srcdst = jnp.concatenate([edge_index, jnp.stack([padi, padi], 0)], 1)
    e2 = srcdst.reshape(2, NCH2, CH).transpose(1, 0, 2)
    mmp = jnp.pad(motif_mask, ((0, 0), (0, pade)))
    e5 = jnp.concatenate([srcdst, mmp], 0).reshape(5, NCH2, CH).transpose(
        1, 0, 2)
    x_pad = jnp.pad(x, ((0, NPAD - N), (0, 0)))
    aw2 = jnp.concatenate([attn_w[:H], attn_w[H:]], axis=1)
    ones_ch = jnp.ones((CH,), jnp.float32)
    zeros1 = jnp.zeros((SLICE,), jnp.float32)
    zrows = jnp.zeros((SLICE, D), jnp.float32)

    degparts = _sc_deg(e2, ones_ch, zeros1).reshape(NC, 2, NPAD)
    xn, ndcol = _tc2(degparts, x_pad)
    aggparts = _sc_agg(xn, e2, zrows)
    z, uv, zsum, muv = _tc4(aggparts, xn, ndcol, conv_w, conv_b[None, :],
                            W_w, W_b[None, :], aw2)
    m_scalar = jnp.maximum(muv[0, 0] + muv[0, 1] + attn_b[0], 0.0)
    m16 = jnp.full((L,), m_scalar, jnp.float32)
    b16 = jnp.full((L,), attn_b[0], jnp.float32)
    u = uv[:, 0]
    v = uv[:, 1]
    denflat = _sc_den(u, v, m16, b16, e5, zeros1)
    wflat = _sc_w(u, v, m16, b16, e5, denflat, zeros1)
    wparts = wflat.reshape(NC, K, NPAD)
    lw4 = lin_w.reshape(K + 1, H, C)
    return _tc7(wparts, z, zsum, Wk_w, Wk_b, lw4, lin_b[None, :])
